# baseline (speedup 1.0000x reference)
import functools

import jax
import jax.numpy as jnp
from jax import lax
from jax.experimental import pallas as pl
from jax.experimental.pallas import tpu as pltpu
from jax.experimental.pallas import tpu_sc as plsc

_N = 10000
_NP = 10240
_E = 320000
_EPAD = 327680

_NC = 2
_NS = 16
_NW = _NC * _NS
_EPW = _EPAD // _NW
_NB = 32
_BROWS = _NP // _NB
_CAP = 544
_CK = 64
_CMAX = 512

_MAGIC = 13108


def _lin_body(x_ref, w_ref, b_ref, o_ref):
    o_ref[...] = (
        jnp.dot(x_ref[...], w_ref[...], preferred_element_type=jnp.float32)
        + b_ref[...]
    )


def _mid_body(p_ref, w_ref, b_ref, o_ref):
    h = jnp.maximum(p_ref[...], 0.0)
    o_ref[...] = (
        jnp.dot(h, w_ref[...], preferred_element_type=jnp.float32) + b_ref[...]
    )


def _argmax_body(p_ref, o_ref):
    h = p_ref[...]
    col = lax.broadcasted_iota(jnp.int32, h.shape, 1)
    h = jnp.where(col < 40, h, -jnp.inf)
    m = jnp.max(h, axis=1, keepdims=True)
    idx = jnp.min(jnp.where(h >= m, col, jnp.int32(2**30)), axis=1)
    o_ref[...] = idx[:, None]


_BR = 2048


def _tc_first(x, W, b):
    n, k = x.shape
    m = W.shape[1]
    return pl.pallas_call(
        _lin_body,
        grid=(n // _BR,),
        in_specs=[
            pl.BlockSpec((_BR, k), lambda i: (i, 0)),
            pl.BlockSpec((k, m), lambda i: (0, 0)),
            pl.BlockSpec((1, m), lambda i: (0, 0)),
        ],
        out_specs=pl.BlockSpec((_BR, m), lambda i: (i, 0)),
        out_shape=jax.ShapeDtypeStruct((n, m), jnp.float32),
    )(x, W, b)


def _tc_mid(p, W, b):
    n, k = p.shape
    m = W.shape[1]
    return pl.pallas_call(
        _mid_body,
        grid=(n // _BR,),
        in_specs=[
            pl.BlockSpec((_BR, k), lambda i: (i, 0)),
            pl.BlockSpec((k, m), lambda i: (0, 0)),
            pl.BlockSpec((1, m), lambda i: (0, 0)),
        ],
        out_specs=pl.BlockSpec((_BR, m), lambda i: (i, 0)),
        out_shape=jax.ShapeDtypeStruct((n, m), jnp.float32),
    )(p, W, b)


def _tc_argmax(p):
    n, k = p.shape
    return pl.pallas_call(
        _argmax_body,
        grid=(n // _BR,),
        in_specs=[pl.BlockSpec((_BR, k), lambda i: (i, 0))],
        out_specs=pl.BlockSpec((_BR, 1), lambda i: (i, 0)),
        out_shape=jax.ShapeDtypeStruct((n, 1), jnp.int32),
    )(p)


def _bucket_body(src_hbm, dst_hbm, ev_hbm, rs_hbm, rd_hbm, re_hbm, cnt_hbm,
                 sv, dv, ev, s_slab, d_slab, e_slab, cbuf, cnt, sem):
    c = lax.axis_index("c")
    s = lax.axis_index("s")
    wid = c * _NS + s
    base = wid * _EPW
    pltpu.sync_copy(src_hbm.at[pl.ds(base, _EPW)], sv)
    pltpu.sync_copy(dst_hbm.at[pl.ds(base, _EPW)], dv)
    pltpu.sync_copy(ev_hbm.at[pl.ds(base, _EPW)], ev)

    zi = jnp.zeros((16,), jnp.int32)
    zf = jnp.zeros((16,), jnp.float32)

    def zb(r, carry):
        for j in range(_CAP // 16):
            sl = pl.ds(j * 16, 16)
            s_slab[r, sl] = zi
            d_slab[r, sl] = zi
            e_slab[r, sl] = zf
        return carry

    lax.fori_loop(0, _NB, zb, 0)
    for b in range(_NB):
        cnt[b] = 0
    iota = lax.iota(jnp.int32, 16)

    def grp(g, carry):
        o = g * 16
        dvv = dv[pl.ds(o, 16)]
        svv = sv[pl.ds(o, 16)]
        evv = ev[pl.ds(o, 16)]
        bv = jnp.right_shift(dvv * _MAGIC, 22)
        dloc = dvv - bv * _BROWS
        for l in range(16):
            bl = bv[l]
            cb = jnp.minimum(cnt[bl], _CAP - 1)
            m = iota == l
            ir = jnp.broadcast_to(bl, (16,))
            ic = jnp.broadcast_to(cb, (16,))
            plsc.store_scatter(s_slab, [ir, ic], svv, mask=m)
            plsc.store_scatter(d_slab, [ir, ic], dloc, mask=m)
            plsc.store_scatter(e_slab, [ir, ic], evv, mask=m)
            cnt[bl] = cb + 1
        return carry

    lax.fori_loop(0, _EPW // 16, grp, 0)

    v0 = jnp.zeros((16,), jnp.int32)
    v1 = jnp.zeros((16,), jnp.int32)
    for b in range(16):
        v0 = jnp.where(iota == b, jnp.broadcast_to(cnt[b], (16,)), v0)
        v1 = jnp.where(iota == b, jnp.broadcast_to(cnt[b + 16], (16,)), v1)
    cbuf[pl.ds(0, 16)] = v0
    cbuf[pl.ds(16, 16)] = v1
    pltpu.sync_copy(cbuf, cnt_hbm.at[pl.ds(wid * _NB, _NB)])

    def drain(b, carry):
        pltpu.sync_copy(s_slab.at[b], rs_hbm.at[b, wid])
        pltpu.sync_copy(d_slab.at[b], rd_hbm.at[b, wid])
        pltpu.sync_copy(e_slab.at[b], re_hbm.at[b, wid])
        return carry

    lax.fori_loop(0, _NB, drain, 0)


def _bucketize(src, dst, ev):
    mesh = plsc.VectorSubcoreMesh(core_axis_name="c", subcore_axis_name="s")
    f = pl.kernel(
        _bucket_body,
        out_type=(
            jax.ShapeDtypeStruct((_NB, _NW, _CAP), jnp.int32),
            jax.ShapeDtypeStruct((_NB, _NW, _CAP), jnp.int32),
            jax.ShapeDtypeStruct((_NB, _NW, _CAP), jnp.float32),
            jax.ShapeDtypeStruct((_NW * _NB,), jnp.int32),
        ),
        mesh=mesh,
        scratch_types=[
            pltpu.VMEM((_EPW,), jnp.int32),
            pltpu.VMEM((_EPW,), jnp.int32),
            pltpu.VMEM((_EPW,), jnp.float32),
            pltpu.VMEM((_NB, _CAP), jnp.int32),
            pltpu.VMEM((_NB, _CAP), jnp.int32),
            pltpu.VMEM((_NB, _CAP), jnp.float32),
            pltpu.VMEM((_NB,), jnp.int32),
            pltpu.SMEM((_NB,), jnp.int32),
            pltpu.SemaphoreType.DMA,
        ],
        compiler_params=pltpu.CompilerParams(
            use_tc_tiling_on_sc=False, needs_layout_passes=False),
    )
    return f(src, dst, ev)


def _spmm_body(z_hbm, rs_hbm, rd_hbm, re_hbm, cnt_hbm, out_hbm,
               srcb, dstb, evbf, rows, cntv, acc, sem):
    F = rows.shape[1]
    c = lax.axis_index("c")
    s = lax.axis_index("s")
    wid = c * _NS + s

    zf = jnp.zeros((16,), jnp.float32)

    def za(r, carry):
        for j in range(F // 16):
            acc[r, pl.ds(j * 16, 16)] = zf
        return carry

    lax.fori_loop(0, _BROWS, za, 0)

    pltpu.sync_copy(cnt_hbm, cntv.at[pl.ds(0, _NW * _NB)])
    pltpu.sync_copy(rs_hbm.at[wid], srcb)
    pltpu.sync_copy(rd_hbm.at[wid], dstb)
    pltpu.sync_copy(re_hbm.at[wid], evbf)

    def prod(p, carry):
        cv = cntv[pl.ds(p * _NB + wid, 16)]
        cn = jnp.minimum(cv[0], _CMAX)
        nck = (cn + _CK - 1) // _CK

        def chunk(k, carry2):
            pltpu.async_copy(
                z_hbm.at[srcb.at[p, pl.ds(k * _CK, _CK)]], rows, sem).wait()

            def edge(e, carry3):
                slot = k * _CK + e
                evx = jnp.broadcast_to(evbf[p, pl.ds(slot, 16)][0], (16,))
                d = dstb[p, pl.ds(slot, 16)][0]
                for j in range(F // 16):
                    sl = pl.ds(j * 16, 16)
                    plsc.addupdate(acc.at[d, sl], rows[e, sl] * evx)
                return carry3

            lax.fori_loop(0, _CK, edge, 0)
            return carry2

        lax.fori_loop(0, nck, chunk, 0)
        return carry

    lax.fori_loop(0, _NW, prod, 0)
    pltpu.sync_copy(acc, out_hbm.at[pl.ds(wid * _BROWS, _BROWS)])


def _spmm(z, rs, rd, re, cnt):
    n, F = z.shape
    mesh = plsc.VectorSubcoreMesh(core_axis_name="c", subcore_axis_name="s")
    f = pl.kernel(
        _spmm_body,
        out_type=jax.ShapeDtypeStruct((n, F), jnp.float32),
        mesh=mesh,
        scratch_types=[
            pltpu.VMEM((_NW, _CAP), jnp.int32),
            pltpu.VMEM((_NW, _CAP), jnp.int32),
            pltpu.VMEM((_NW, _CAP), jnp.float32),
            pltpu.VMEM((_CK, F), jnp.float32),
            pltpu.VMEM((_NW * _NB + 16,), jnp.int32),
            pltpu.VMEM((_BROWS, F), jnp.float32),
            pltpu.SemaphoreType.DMA,
        ],
        compiler_params=pltpu.CompilerParams(use_tc_tiling_on_sc=False),
    )
    return f(z, rs, rd, re, cnt)


def kernel(x, edge_index, edge_vals, W1, b1, W2, b2, W3, b3):
    pad_e = _EPAD - _E
    dpad = (jnp.arange(pad_e, dtype=jnp.int32) % _NB) * _BROWS
    dst = jnp.concatenate([edge_index[0], dpad])
    src = jnp.pad(edge_index[1], (0, pad_e))
    evp = jnp.pad(edge_vals, (0, pad_e))

    rs, rd, re, cnt = _bucketize(src, dst, evp)

    xp = jnp.pad(x, ((0, _NP - _N), (0, 0)))
    z1 = _tc_first(xp, W1, b1.reshape(1, -1))
    p1 = _spmm(z1, rs, rd, re, cnt)
    z2 = _tc_mid(p1, W2, b2.reshape(1, -1))
    p2 = _spmm(z2, rs, rd, re, cnt)
    W3p = jnp.pad(W3, ((0, 0), (0, 8)))
    b3p = jnp.pad(b3, (0, 8)).reshape(1, -1)
    z3 = _tc_mid(p2, W3p, b3p)
    p3 = _spmm(z3, rs, rd, re, cnt)
    out = _tc_argmax(p3)
    return out[:_N, 0]


# speedup vs baseline: 2.0067x; 2.0067x over previous
import functools

import jax
import jax.numpy as jnp
from jax import lax
from jax.experimental import pallas as pl
from jax.experimental.pallas import tpu as pltpu
from jax.experimental.pallas import tpu_sc as plsc

_N = 10000
_NP = 10240
_E = 320000
_EPAD = 327680

_NC = 2
_NS = 16
_NW = _NC * _NS
_EPW = _EPAD // _NW
_CH = 128
_NCHUNK = _EPW // _CH
_RPT = _NP // _NS
_ZR = 32


def _lin_body(x_ref, w_ref, b_ref, o_ref):
    o_ref[...] = (
        jnp.dot(x_ref[...], w_ref[...], preferred_element_type=jnp.float32)
        + b_ref[...]
    )


def _mid_body(p0_ref, p1_ref, w_ref, b_ref, o_ref):
    h = jnp.maximum(p0_ref[...] + p1_ref[...], 0.0)
    o_ref[...] = (
        jnp.dot(h, w_ref[...], preferred_element_type=jnp.float32) + b_ref[...]
    )


def _argmax_body(p0_ref, p1_ref, o_ref):
    h = p0_ref[...] + p1_ref[...]
    col = lax.broadcasted_iota(jnp.int32, h.shape, 1)
    valid = col < 40
    h = jnp.where(valid, h, -jnp.inf)
    m = jnp.max(h, axis=1, keepdims=True)
    idx = jnp.min(jnp.where(h >= m, col, jnp.int32(2**30)), axis=1)
    o_ref[...] = idx[:, None]


_BR = 2048


def _bcast_body(e_ref, o_ref):
    o_ref[...] = jnp.broadcast_to(e_ref[...], o_ref.shape)


def _tc_ev_bcast(ev):
    e2 = ev.reshape(_EPAD, 1)
    bre = 4096
    return pl.pallas_call(
        _bcast_body,
        grid=(_EPAD // bre,),
        in_specs=[pl.BlockSpec((bre, 1), lambda i: (i, 0))],
        out_specs=pl.BlockSpec((bre, 16), lambda i: (i, 0)),
        out_shape=jax.ShapeDtypeStruct((_EPAD, 16), jnp.float32),
    )(e2)


def _tc_first(x, W, b):
    n, k = x.shape
    m = W.shape[1]
    return pl.pallas_call(
        _lin_body,
        grid=(n // _BR,),
        in_specs=[
            pl.BlockSpec((_BR, k), lambda i: (i, 0)),
            pl.BlockSpec((k, m), lambda i: (0, 0)),
            pl.BlockSpec((1, m), lambda i: (0, 0)),
        ],
        out_specs=pl.BlockSpec((_BR, m), lambda i: (i, 0)),
        out_shape=jax.ShapeDtypeStruct((n, m), jnp.float32),
    )(x, W, b)


def _tc_mid(p0, p1, W, b):
    n, k = p0.shape
    m = W.shape[1]
    return pl.pallas_call(
        _mid_body,
        grid=(n // _BR,),
        in_specs=[
            pl.BlockSpec((_BR, k), lambda i: (i, 0)),
            pl.BlockSpec((_BR, k), lambda i: (i, 0)),
            pl.BlockSpec((k, m), lambda i: (0, 0)),
            pl.BlockSpec((1, m), lambda i: (0, 0)),
        ],
        out_specs=pl.BlockSpec((_BR, m), lambda i: (i, 0)),
        out_shape=jax.ShapeDtypeStruct((n, m), jnp.float32),
    )(p0, p1, W, b)


def _tc_argmax(p0, p1):
    n, k = p0.shape
    return pl.pallas_call(
        _argmax_body,
        grid=(n // _BR,),
        in_specs=[
            pl.BlockSpec((_BR, k), lambda i: (i, 0)),
            pl.BlockSpec((_BR, k), lambda i: (i, 0)),
        ],
        out_specs=pl.BlockSpec((_BR, 1), lambda i: (i, 0)),
        out_shape=jax.ShapeDtypeStruct((n, 1), jnp.int32),
    )(p0, p1)


def _spmm_sc_body(z_hbm, src_hbm, dst_hbm, ev_hbm, out_hbm,
                  src_all, dst_all, evb, rows, zbuf, acc, sem):
    F = rows.shape[1]
    c = lax.axis_index("c")
    s = lax.axis_index("s")
    wid = c * _NS + s

    pltpu.sync_copy(src_hbm.at[pl.ds(wid * _NCHUNK, _NCHUNK)], src_all)
    pltpu.sync_copy(dst_hbm.at[pl.ds(wid * _NCHUNK, _NCHUNK)], dst_all)

    zero = jnp.zeros((16,), jnp.float32)
    for r in range(_ZR):
        for j in range(F // 16):
            zbuf[r, pl.ds(j * 16, 16)] = zero

    def zero_acc(i, carry):
        pltpu.sync_copy(zbuf, acc.at[pl.ds(s * _RPT + i * _ZR, _ZR)])
        return carry

    lax.fori_loop(0, _RPT // _ZR, zero_acc, 0)
    plsc.subcore_barrier()

    def chunk_body(k, carry):
        row = wid * _NCHUNK + k
        c_ev = pltpu.async_copy(ev_hbm.at[row], evb, sem)
        c_g = pltpu.async_copy(z_hbm.at[src_all.at[k]], rows, sem)
        c_ev.wait()
        c_g.wait()

        def scale_body(e, c2):
            evx = evb[e, pl.ds(0, 16)]
            for j in range(F // 16):
                sl = pl.ds(j * 16, 16)
                rows[e, sl] = rows[e, sl] * evx
            return c2

        lax.fori_loop(0, _CH, scale_body, 0)
        pltpu.sync_copy(rows, acc.at[dst_all.at[k]], add=True)
        return carry

    lax.fori_loop(0, _NCHUNK, chunk_body, 0)
    plsc.subcore_barrier()

    pltpu.sync_copy(acc.at[pl.ds(s * _RPT, _RPT)],
                    out_hbm.at[c, pl.ds(s * _RPT, _RPT)])


def _spmm(z, src, dst, ev):
    n, F = z.shape
    mesh = plsc.VectorSubcoreMesh(core_axis_name="c", subcore_axis_name="s")
    f = pl.kernel(
        _spmm_sc_body,
        out_type=jax.ShapeDtypeStruct((_NC, n, F), jnp.float32),
        mesh=mesh,
        scratch_types=[
            pltpu.VMEM((_NCHUNK, _CH), jnp.int32),
            pltpu.VMEM((_NCHUNK, _CH), jnp.int32),
            pltpu.VMEM((_CH, 16), jnp.float32),
            pltpu.VMEM((_CH, F), jnp.float32),
            pltpu.VMEM((_ZR, F), jnp.float32),
            pltpu.VMEM_SHARED((n, F), jnp.float32),
            pltpu.SemaphoreType.DMA,
        ],
        compiler_params=pltpu.CompilerParams(use_tc_tiling_on_sc=False),
    )
    return f(z, src, dst, ev)


def kernel(x, edge_index, edge_vals, W1, b1, W2, b2, W3, b3):
    pad_e = _EPAD - _E
    dst = jnp.pad(edge_index[0], (0, pad_e)).reshape(_NW * _NCHUNK, _CH)
    src = jnp.pad(edge_index[1], (0, pad_e)).reshape(_NW * _NCHUNK, _CH)
    evp = jnp.pad(edge_vals, (0, pad_e))

    xp = jnp.pad(x, ((0, _NP - _N), (0, 0)))
    z1 = _tc_first(xp, W1, b1.reshape(1, -1))
    evb = _tc_ev_bcast(evp).reshape(_NW * _NCHUNK, _CH, 16)
    p1 = _spmm(z1, src, dst, evb)
    z2 = _tc_mid(p1[0], p1[1], W2, b2.reshape(1, -1))
    p2 = _spmm(z2, src, dst, evb)
    W3p = jnp.pad(W3, ((0, 0), (0, 8)))
    b3p = jnp.pad(b3, (0, 8)).reshape(1, -1)
    z3 = _tc_mid(p2[0], p2[1], W3p, b3p)
    p3 = _spmm(z3, src, dst, evb)
    out = _tc_argmax(p3[0], p3[1])
    return out[:_N, 0]
